# baseline (speedup 1.0000x reference)
import jax
import jax.numpy as jnp
from jax import lax
from jax.experimental import pallas as pl

NUM_CLASSES = 1000
ROWS_PER_BLOCK = 128


def _onehot_block(x_ref, o_ref):
    x = x_ref[...]
    k = lax.broadcasted_iota(jnp.int32, o_ref.shape, 2)
    o_ref[...] = (x[:, :, None] == k).astype(jnp.float32)


def kernel(X):
    n, m = X.shape
    grid = (n // ROWS_PER_BLOCK,)
    return pl.pallas_call(
        _onehot_block,
        grid=grid,
        in_specs=[pl.BlockSpec((ROWS_PER_BLOCK, m), lambda i: (i, 0))],
        out_specs=pl.BlockSpec((ROWS_PER_BLOCK, m, NUM_CLASSES), lambda i: (i, 0, 0)),
        out_shape=jax.ShapeDtypeStruct((n, m, NUM_CLASSES), jnp.float32),
    )(X)


# speedup vs baseline: 1.0030x; 1.0030x over previous
import jax
import jax.numpy as jnp
from jax import lax
from jax.experimental import pallas as pl
from jax.experimental.pallas import tpu as pltpu

NUM_CLASSES = 1000
ROWS_PER_BLOCK = 32
NSLOTS = 4


def _onehot_block(x_ref, o_hbm, buf, sems):
    i = pl.program_id(0)
    n = pl.num_programs(0)
    slot = lax.rem(i, NSLOTS)

    @pl.when(i >= NSLOTS)
    def _():
        pltpu.make_async_copy(
            buf.at[slot], o_hbm.at[pl.ds(0, ROWS_PER_BLOCK)], sems.at[slot]
        ).wait()

    x = x_ref[...]
    k = lax.broadcasted_iota(jnp.int32, (ROWS_PER_BLOCK, x_ref.shape[1], NUM_CLASSES), 2)
    buf[slot] = (x[:, :, None] == k).astype(jnp.float32)

    pltpu.make_async_copy(
        buf.at[slot],
        o_hbm.at[pl.ds(i * ROWS_PER_BLOCK, ROWS_PER_BLOCK)],
        sems.at[slot],
    ).start()

    @pl.when(i == n - 1)
    def _():
        for j in range(NSLOTS):
            s = lax.rem(i - j, NSLOTS)
            pltpu.make_async_copy(
                buf.at[s], o_hbm.at[pl.ds(0, ROWS_PER_BLOCK)], sems.at[s]
            ).wait()


def kernel(X):
    n, m = X.shape
    grid = (n // ROWS_PER_BLOCK,)
    return pl.pallas_call(
        _onehot_block,
        grid=grid,
        in_specs=[pl.BlockSpec((ROWS_PER_BLOCK, m), lambda i: (i, 0))],
        out_specs=pl.BlockSpec(memory_space=pltpu.MemorySpace.HBM),
        out_shape=jax.ShapeDtypeStruct((n, m, NUM_CLASSES), jnp.float32),
        scratch_shapes=[
            pltpu.MemorySpace.VMEM(
                (NSLOTS, ROWS_PER_BLOCK, m, NUM_CLASSES), jnp.float32
            ),
            pltpu.SemaphoreType.DMA((NSLOTS,)),
        ],
    )(X)
